# Initial kernel scaffold; baseline (speedup 1.0000x reference)
import jax
import jax.numpy as jnp
from jax.experimental import pallas as pl


def kernel(adj_matrix, node_embs, W):
    raise NotImplementedError("write your pallas kernel here")


# speedup vs baseline: 1.0055x; 1.0055x over previous
import jax
import jax.numpy as jnp
from jax.experimental import pallas as pl
from jax.experimental.pallas import tpu as pltpu


def _gcn_block_kernel(x_ref, w_ref, adj_ref, out_ref, s_ref):
    i = pl.program_id(0)

    @pl.when(i == 0)
    def _compute_support():
        s = jnp.dot(x_ref[...], w_ref[...], preferred_element_type=jnp.float32)
        s_ref[...] = s.astype(jnp.bfloat16)

    a = adj_ref[...].astype(jnp.bfloat16)
    acc = jnp.dot(a, s_ref[...], preferred_element_type=jnp.float32)
    out_ref[...] = jnp.where(acc >= 0, acc, 0.01 * acc)


def kernel(adj_matrix, node_embs, W):
    n_dst, n_src = adj_matrix.shape
    d_in = node_embs.shape[1]
    d_out = W.shape[1]

    bm = 200 if n_dst % 200 == 0 else 8
    grid = (pl.cdiv(n_dst, bm),)

    return pl.pallas_call(
        _gcn_block_kernel,
        grid=grid,
        in_specs=[
            pl.BlockSpec((n_src, d_in), lambda i: (0, 0)),
            pl.BlockSpec((d_in, d_out), lambda i: (0, 0)),
            pl.BlockSpec((bm, n_src), lambda i: (i, 0)),
        ],
        out_specs=pl.BlockSpec((bm, d_out), lambda i: (i, 0)),
        out_shape=jax.ShapeDtypeStruct((n_dst, d_out), jnp.float32),
        scratch_shapes=[pltpu.VMEM((n_src, d_out), jnp.bfloat16)],
        compiler_params=pltpu.CompilerParams(
            dimension_semantics=("arbitrary",),
        ),
    )(node_embs, W, adj_matrix)


# speedup vs baseline: 1.0087x; 1.0032x over previous
import jax
import jax.numpy as jnp
from jax.experimental import pallas as pl
from jax.experimental.pallas import tpu as pltpu


def _gcn_block_kernel(x_ref, w_ref, adj_ref, out_ref, s_ref):
    i = pl.program_id(0)

    @pl.when(i == 0)
    def _compute_support():
        s_ref[...] = jnp.dot(
            x_ref[...], w_ref[...], preferred_element_type=jnp.float32
        )

    acc = jnp.dot(adj_ref[...], s_ref[...], preferred_element_type=jnp.float32)
    out_ref[...] = jnp.where(acc >= 0, acc, 0.01 * acc)


def kernel(adj_matrix, node_embs, W):
    n_dst, n_src = adj_matrix.shape
    d_in = node_embs.shape[1]
    d_out = W.shape[1]

    bm = 200 if n_dst % 200 == 0 else 8
    grid = (pl.cdiv(n_dst, bm),)

    return pl.pallas_call(
        _gcn_block_kernel,
        grid=grid,
        in_specs=[
            pl.BlockSpec((n_src, d_in), lambda i: (0, 0)),
            pl.BlockSpec((d_in, d_out), lambda i: (0, 0)),
            pl.BlockSpec((bm, n_src), lambda i: (i, 0)),
        ],
        out_specs=pl.BlockSpec((bm, d_out), lambda i: (i, 0)),
        out_shape=jax.ShapeDtypeStruct((n_dst, d_out), jnp.float32),
        scratch_shapes=[pltpu.VMEM((n_src, d_out), jnp.float32)],
        compiler_params=pltpu.CompilerParams(
            dimension_semantics=("arbitrary",),
        ),
    )(node_embs, W, adj_matrix)


# speedup vs baseline: 1.0181x; 1.0093x over previous
import jax
import jax.numpy as jnp
from jax.experimental import pallas as pl
from jax.experimental.pallas import tpu as pltpu


def _gcn_block_kernel(x_ref, w_ref, adj_ref, out_ref, s_ref):
    i = pl.program_id(0)

    @pl.when(i == 0)
    def _compute_support():
        s_ref[...] = jnp.dot(
            x_ref[...], w_ref[...], preferred_element_type=jnp.float32
        )

    acc = jnp.dot(adj_ref[...], s_ref[...], preferred_element_type=jnp.float32)
    out_ref[...] = jnp.where(acc >= 0, acc, 0.01 * acc)


def kernel(adj_matrix, node_embs, W):
    n_dst, n_src = adj_matrix.shape
    d_in = node_embs.shape[1]
    d_out = W.shape[1]

    bm = 400 if n_dst % 400 == 0 else 8
    grid = (pl.cdiv(n_dst, bm),)

    return pl.pallas_call(
        _gcn_block_kernel,
        grid=grid,
        in_specs=[
            pl.BlockSpec((n_src, d_in), lambda i: (0, 0)),
            pl.BlockSpec((d_in, d_out), lambda i: (0, 0)),
            pl.BlockSpec((bm, n_src), lambda i: (i, 0)),
        ],
        out_specs=pl.BlockSpec((bm, d_out), lambda i: (i, 0)),
        out_shape=jax.ShapeDtypeStruct((n_dst, d_out), jnp.float32),
        scratch_shapes=[pltpu.VMEM((n_src, d_out), jnp.float32)],
        compiler_params=pltpu.CompilerParams(
            dimension_semantics=("arbitrary",),
        ),
    )(node_embs, W, adj_matrix)
